# Initial kernel scaffold; baseline (speedup 1.0000x reference)
import jax
import jax.numpy as jnp
from jax.experimental import pallas as pl


def kernel(input_ids, token_type_ids, word_table, pos_table, type_table, ln_gamma, ln_beta):
    raise NotImplementedError("write your pallas kernel here")


# speedup vs baseline: 2.6627x; 2.6627x over previous
import functools

import jax
import jax.numpy as jnp
from jax import lax
from jax.experimental import pallas as pl
from jax.experimental.pallas import tpu as pltpu
from jax.experimental.pallas import tpu_sc as plsc

_EPS = 1e-5
_C = 64
_BS = 512


@functools.lru_cache(maxsize=None)
def _make_sc_gather(V, D, N):
    info = plsc.get_sparse_core_info()
    NC, NS = info.num_cores, info.num_subcores
    NW = NC * NS
    tpw = N // NW
    nchunks = tpw // _C

    mesh = plsc.VectorSubcoreMesh(core_axis_name="c", subcore_axis_name="s")

    @functools.partial(
        pl.kernel,
        mesh=mesh,
        out_type=jax.ShapeDtypeStruct((N, D), jnp.float32),
        scratch_types=[
            pltpu.VMEM((nchunks, _C), jnp.int32),
            pltpu.VMEM((_C, D), jnp.float32),
            pltpu.VMEM((_C, D), jnp.float32),
            pltpu.SemaphoreType.DMA,
            pltpu.SemaphoreType.DMA,
            pltpu.SemaphoreType.DMA,
            pltpu.SemaphoreType.DMA,
        ],
    )
    def gather_k(table_hbm, idx_hbm, out_hbm, idx_v, buf0, buf1, g0, g1, o0, o1):
        wid = lax.axis_index("s") * NC + lax.axis_index("c")
        base = wid * tpw
        bufs = (buf0, buf1)
        gsems = (g0, g1)
        osems = (o0, o1)

        pltpu.sync_copy(idx_hbm.at[pl.ds(wid * nchunks, nchunks)], idx_v)

        def gather(c, slot):
            return pltpu.make_async_copy(
                table_hbm.at[idx_v.at[c]], bufs[slot], gsems[slot])

        def out_copy(c, slot):
            return pltpu.make_async_copy(
                bufs[slot], out_hbm.at[pl.ds(base + c * _C, _C)], osems[slot])

        gather(0, 0).start()
        for c in range(nchunks):
            cur = c % 2
            gather(c, cur).wait()
            out_copy(c, cur).start()
            if c + 1 < nchunks:
                if c >= 1:
                    out_copy(c - 1, (c - 1) % 2).wait()
                gather(c + 1, (c + 1) % 2).start()
        out_copy(nchunks - 2, (nchunks - 2) % 2).wait()
        out_copy(nchunks - 1, (nchunks - 1) % 2).wait()

    return gather_k


@functools.lru_cache(maxsize=None)
def _make_tc_ln(B, S, D):
    grid = (S // _BS, B)

    def body(g_ref, p_ref, tt_ref, ttab_ref, gam_ref, bet_ref, o_ref):
        e = g_ref[0] + p_ref[...]
        t0 = ttab_ref[0]
        t1 = ttab_ref[1]
        ttf = tt_ref[0, 0, 0, :].astype(jnp.float32)[:, None]
        e = e + (t0[None, :] + ttf * (t1 - t0)[None, :])
        mu = jnp.mean(e, axis=1, keepdims=True)
        d = e - mu
        var = jnp.mean(d * d, axis=1, keepdims=True)
        r = lax.rsqrt(var + _EPS)
        o_ref[0] = (d * r) * gam_ref[0][None, :] + bet_ref[0][None, :]

    return pl.pallas_call(
        body,
        grid=grid,
        in_specs=[
            pl.BlockSpec((1, _BS, D), lambda s, b: (b, s, 0)),
            pl.BlockSpec((_BS, D), lambda s, b: (s, 0)),
            pl.BlockSpec((1, 1, 1, _BS), lambda s, b: (b, s, 0, 0)),
            pl.BlockSpec((2, D), lambda s, b: (0, 0)),
            pl.BlockSpec((1, D), lambda s, b: (0, 0)),
            pl.BlockSpec((1, D), lambda s, b: (0, 0)),
        ],
        out_specs=pl.BlockSpec((1, _BS, D), lambda s, b: (b, s, 0)),
        out_shape=jax.ShapeDtypeStruct((B, S, D), jnp.float32),
    )


def kernel(input_ids, token_type_ids, word_table, pos_table, type_table,
           ln_gamma, ln_beta):
    B, S = input_ids.shape
    V, D = word_table.shape
    N = B * S

    ids2 = input_ids.reshape(N // _C, _C).astype(jnp.int32)
    gathered = _make_sc_gather(V, D, N)(word_table, ids2)
    g3 = gathered.reshape(B, S, D)
    tt4 = token_type_ids.reshape(B, S // _BS, 1, _BS).astype(jnp.int32)
    return _make_tc_ln(B, S, D)(
        g3, pos_table, tt4, type_table,
        ln_gamma.reshape(1, D), ln_beta.reshape(1, D))
